# Initial kernel scaffold; baseline (speedup 1.0000x reference)
import jax
import jax.numpy as jnp
from jax.experimental import pallas as pl


def kernel(x, edge_index_local, edge_index_global, W_local, att_src_local, att_dst_local, bias_local, W_global, att_src_global, att_dst_global, bias_global, W_comb, b_comb):
    raise NotImplementedError("write your pallas kernel here")


# speedup vs baseline: 13.2506x; 13.2506x over previous
import functools

import jax
import jax.numpy as jnp
from jax import lax
from jax.experimental import pallas as pl
from jax.experimental.pallas import tpu as pltpu
from jax.experimental.pallas import tpu_sc as plsc

NN = 10000
NPAD = 10240
TRASH = NPAD - 1
D = 128
DA = 144
BLK = 512
EL_PAD = 327680
EG_PAD = 163840
CH_L = 80
CH_G = 40
NTILE = 32
RPT = NPAD // 16


def _prep_body(x_ref, wl_ref, wg_ref, att_ref, xwal_ref, xwag_ref, s_ref):
    xb = x_ref[...]
    xwl = jnp.dot(xb, wl_ref[...], preferred_element_type=jnp.float32)
    xwg = jnp.dot(xb, wg_ref[...], preferred_element_type=jnp.float32)
    att = att_ref[...]
    lane = lax.broadcasted_iota(jnp.int32, (BLK, DA - D), 1)
    sl_col = jnp.sum(xwl * att[0:1, :], axis=1, keepdims=True)
    sg_col = jnp.sum(xwg * att[2:3, :], axis=1, keepdims=True)
    zero = jnp.zeros((BLK, DA - D), jnp.float32)
    aug_l = jnp.where(lane == 0, 1.0, jnp.where(lane == 1, sl_col + zero, 0.0))
    aug_g = jnp.where(lane == 0, 1.0, jnp.where(lane == 1, sg_col + zero, 0.0))
    xwal_ref[...] = jnp.concatenate([xwl, aug_l], axis=1)
    xwag_ref[...] = jnp.concatenate([xwg, aug_g], axis=1)
    dn = (((1,), (1,)), ((), ()))
    sl = lax.dot_general(att, xwl, dn, preferred_element_type=jnp.float32)
    sg = lax.dot_general(att, xwg, dn, preferred_element_type=jnp.float32)
    rows = lax.broadcasted_iota(jnp.int32, (8, BLK), 0)
    s_ref[...] = jnp.where(rows < 2, sl, sg)


def _precompute(x_pad, W_l, W_g, att8):
    nblk = NPAD // BLK
    return pl.pallas_call(
        _prep_body,
        grid=(nblk,),
        in_specs=[
            pl.BlockSpec((BLK, D), lambda i: (i, 0)),
            pl.BlockSpec((D, D), lambda i: (0, 0)),
            pl.BlockSpec((D, D), lambda i: (0, 0)),
            pl.BlockSpec((8, D), lambda i: (0, 0)),
        ],
        out_specs=[
            pl.BlockSpec((BLK, DA), lambda i: (i, 0)),
            pl.BlockSpec((BLK, DA), lambda i: (i, 0)),
            pl.BlockSpec((8, BLK), lambda i: (0, i)),
        ],
        out_shape=[
            jax.ShapeDtypeStruct((NPAD, DA), jnp.float32),
            jax.ShapeDtypeStruct((NPAD, DA), jnp.float32),
            jax.ShapeDtypeStruct((8, NPAD), jnp.float32),
        ],
    )(x_pad, W_l, W_g, att8)


def _sc_body(xwal, xwag, s_hbm, edl, edg, out_l, out_g,
             d_v, ed_c, w_v, rows_v, acc, sem):
    cid = lax.axis_index("c")
    sid = lax.axis_index("s")
    wid = sid * 2 + cid

    def zero_rows():
        def body(j, _):
            for l in range(DA // 16):
                rows_v[j, pl.ds(l * 16, 16)] = jnp.zeros((16,), jnp.float32)
            return 0
        lax.fori_loop(0, 128, body, 0)

    def phase(ed, nch, xwa, out_hbm, drow):
        pltpu.sync_copy(s_hbm.at[drow], d_v)
        zero_rows()
        for k in range(RPT // 128):
            pltpu.sync_copy(rows_v, acc.at[pl.ds(sid * RPT + k * 128, 128)])
        plsc.subcore_barrier()

        def chunk(c, _):
            pltpu.sync_copy(ed.at[wid, c], ed_c)
            pltpu.async_copy(xwa.at[ed_c.at[0]], rows_v, sem).wait()
            for g in range(8):
                ridx = g * 16 + lax.iota(jnp.int32, 16)
                cidx = jnp.full((16,), D + 1, jnp.int32)
                s16 = plsc.load_gather(rows_v, [ridx, cidx])
                d16 = plsc.load_gather(d_v, [ed_c[1, pl.ds(g * 16, 16)]])
                t = s16 + d16
                t = jnp.where(t >= 0.0, t, 0.2 * t)
                w_v[pl.ds(g * 16, 16)] = jnp.exp(t)

            def scale(j, _):
                wj = plsc.load_gather(w_v, [jnp.zeros((16,), jnp.int32) + j])
                for l in range(DA // 16):
                    rows_v[j, pl.ds(l * 16, 16)] = rows_v[j, pl.ds(l * 16, 16)] * wj
                return 0
            lax.fori_loop(0, 128, scale, 0)
            pltpu.sync_copy(rows_v, acc.at[ed_c.at[1]], add=True)
            return 0
        lax.fori_loop(0, nch, chunk, 0)
        plsc.subcore_barrier()
        pltpu.sync_copy(acc.at[pl.ds(sid * RPT, RPT)],
                        out_hbm.at[cid, pl.ds(sid * RPT, RPT)])
        plsc.subcore_barrier()

    phase(edl, CH_L, xwal, out_l, 1)
    phase(edg, CH_G, xwag, out_g, 3)


def _sc_accumulate(xwal, xwag, s8, edl, edg):
    mesh = plsc.VectorSubcoreMesh(core_axis_name="c", subcore_axis_name="s")
    return pl.kernel(
        _sc_body,
        out_type=[
            jax.ShapeDtypeStruct((2, NPAD, DA), jnp.float32),
            jax.ShapeDtypeStruct((2, NPAD, DA), jnp.float32),
        ],
        mesh=mesh,
        scratch_types=[
            pltpu.VMEM((NPAD,), jnp.float32),
            pltpu.VMEM((2, 128), jnp.int32),
            pltpu.VMEM((128,), jnp.float32),
            pltpu.VMEM((128, DA), jnp.float32),
            pltpu.VMEM_SHARED((NPAD, DA), jnp.float32),
            pltpu.SemaphoreType.DMA,
        ],
        compiler_params=pltpu.CompilerParams(
            needs_layout_passes=False, use_tc_tiling_on_sc=False
        ),
    )(xwal, xwag, s8, edl, edg)


def _combine_body(outl_ref, outg_ref, aux_ref, wc_ref, y_ref):
    al = outl_ref[0] + outl_ref[1]
    ag = outg_ref[0] + outg_ref[1]
    hl = al[:, :D] / (al[:, D:D + 1] + 1e-16) + aux_ref[0:1, :]
    hg = ag[:, :D] / (ag[:, D:D + 1] + 1e-16) + aux_ref[1:2, :]
    h = jnp.concatenate([hl, hg], axis=1)
    z = jnp.dot(h, wc_ref[...], preferred_element_type=jnp.float32)
    z = z + aux_ref[2:3, :]
    y_ref[...] = jnp.where(z > 0.0, z, jnp.exp(jnp.minimum(z, 0.0)) - 1.0)


def _combine(out_l, out_g, aux, W_comb):
    nblk = NPAD // BLK
    return pl.pallas_call(
        _combine_body,
        grid=(nblk,),
        in_specs=[
            pl.BlockSpec((2, BLK, DA), lambda i: (0, i, 0)),
            pl.BlockSpec((2, BLK, DA), lambda i: (0, i, 0)),
            pl.BlockSpec((8, D), lambda i: (0, 0)),
            pl.BlockSpec((2 * D, D), lambda i: (0, 0)),
        ],
        out_specs=pl.BlockSpec((BLK, D), lambda i: (i, 0)),
        out_shape=jax.ShapeDtypeStruct((NPAD, D), jnp.float32),
    )(out_l, out_g, aux, W_comb)


def kernel(x, edge_index_local, edge_index_global, W_local, att_src_local,
           att_dst_local, bias_local, W_global, att_src_global,
           att_dst_global, bias_global, W_comb, b_comb):
    x_pad = jnp.pad(x, ((0, NPAD - NN), (0, 0)))
    att8 = (
        jnp.zeros((8, D), jnp.float32)
        .at[0].set(att_src_local.reshape(D))
        .at[1].set(att_dst_local.reshape(D))
        .at[2].set(att_src_global.reshape(D))
        .at[3].set(att_dst_global.reshape(D))
    )
    xwal, xwag, s8 = _precompute(x_pad, W_local, W_global, att8)

    def prep(e, epad):
        e = jnp.pad(e, ((0, 0), (0, epad - e.shape[1])), constant_values=TRASH)
        src = e[0].reshape(NTILE, -1, 128)
        dst = e[1].reshape(NTILE, -1, 128)
        return jnp.stack([src, dst], axis=2)

    edl = prep(edge_index_local, EL_PAD)
    edg = prep(edge_index_global, EG_PAD)
    out_l, out_g = _sc_accumulate(xwal, xwag, s8, edl, edg)

    aux = (
        jnp.zeros((8, D), jnp.float32)
        .at[0].set(bias_local)
        .at[1].set(bias_global)
        .at[2].set(b_comb)
    )
    y = _combine(out_l, out_g, aux, W_comb)
    return y[:NN]


# speedup vs baseline: 14.2016x; 1.0718x over previous
import jax
import jax.numpy as jnp
from jax import lax
from jax.experimental import pallas as pl
from jax.experimental.pallas import tpu as pltpu
from jax.experimental.pallas import tpu_sc as plsc

NN = 10000
NPAD = 10240
TRASH = NPAD - 1
D = 128
DA = 144
BLK = 512
CK = 64
GRP = 16
EL_PAD = 327680
EG_PAD = 163840
NG_L = 10
NG_G = 5
NTILE = 32
RPT = NPAD // 16


def _prep_body(x_ref, wl_ref, wg_ref, att_ref, xwal_ref, xwag_ref, dt_ref):
    xb = x_ref[...]
    xwl = jnp.dot(xb, wl_ref[...], preferred_element_type=jnp.float32)
    xwg = jnp.dot(xb, wg_ref[...], preferred_element_type=jnp.float32)
    att = att_ref[...]
    lane = lax.broadcasted_iota(jnp.int32, (BLK, DA - D), 1)
    zero = jnp.zeros((BLK, DA - D), jnp.float32)
    sl = jnp.sum(xwl * att[0:1, :], axis=1, keepdims=True)
    sg = jnp.sum(xwg * att[2:3, :], axis=1, keepdims=True)
    aug_l = jnp.where(lane == 0, 1.0, jnp.where(lane == 1, sl + zero, 0.0))
    aug_g = jnp.where(lane == 0, 1.0, jnp.where(lane == 1, sg + zero, 0.0))
    xwal_ref[...] = jnp.concatenate([xwl, aug_l], axis=1)
    xwag_ref[...] = jnp.concatenate([xwg, aug_g], axis=1)
    dl = jnp.sum(xwl * att[1:2, :], axis=1, keepdims=True)
    dg = jnp.sum(xwg * att[3:4, :], axis=1, keepdims=True)
    lane16 = lax.broadcasted_iota(jnp.int32, (BLK, 16), 1)
    zero16 = jnp.zeros((BLK, 16), jnp.float32)
    dt_ref[...] = jnp.where(
        lane16 == 0, dl + zero16, jnp.where(lane16 == 1, dg + zero16, 0.0)
    )


def _precompute(x_pad, W_l, W_g, att8):
    nblk = NPAD // BLK
    return pl.pallas_call(
        _prep_body,
        grid=(nblk,),
        in_specs=[
            pl.BlockSpec((BLK, D), lambda i: (i, 0)),
            pl.BlockSpec((D, D), lambda i: (0, 0)),
            pl.BlockSpec((D, D), lambda i: (0, 0)),
            pl.BlockSpec((8, D), lambda i: (0, 0)),
        ],
        out_specs=[
            pl.BlockSpec((BLK, DA), lambda i: (i, 0)),
            pl.BlockSpec((BLK, DA), lambda i: (i, 0)),
            pl.BlockSpec((BLK, 16), lambda i: (i, 0)),
        ],
        out_shape=[
            jax.ShapeDtypeStruct((NPAD, DA), jnp.float32),
            jax.ShapeDtypeStruct((NPAD, DA), jnp.float32),
            jax.ShapeDtypeStruct((NPAD, 16), jnp.float32),
        ],
    )(x_pad, W_l, W_g, att8)


def _sc_body(xwal, xwag, dt, edl, edg, out_l, out_g,
             r0, r1, r2, db0, db1, db2, edg_v, w_v, acc,
             gs0, gs1, gs2, ss0, ss1, ss2):
    cid = lax.axis_index("c")
    sid = lax.axis_index("s")
    wid = sid * 2 + cid
    rows = (r0, r1, r2)
    dbs = (db0, db1, db2)
    gsems = (gs0, gs1, gs2)
    ssems = (ss0, ss1, ss2)

    def phase(ed, ngrp, xwa, out_hbm, dcol):
        @plsc.parallel_loop(0, CK)
        def _(j):
            for l in range(DA // 16):
                r0[j, pl.ds(l * 16, 16)] = jnp.zeros((16,), jnp.float32)
        for k in range(RPT // CK):
            pltpu.sync_copy(r0, acc.at[pl.ds(sid * RPT + k * CK, CK)])
        plsc.subcore_barrier()

        def group(g, _):
            pltpu.sync_copy(ed.at[wid, g], edg_v)

            def gather(k):
                b = k % 3
                rd = pltpu.async_copy(xwa.at[edg_v.at[k, 0]], rows[b], gsems[b])
                dd = pltpu.async_copy(dt.at[edg_v.at[k, 1]], dbs[b], gsems[b])
                return (rd, dd)

            gds = [None] * GRP
            sds = [None] * GRP
            gds[0] = gather(0)
            gds[1] = gather(1)
            for k in range(GRP):
                b = k % 3
                r, dbuf = rows[b], dbs[b]
                gds[k][0].wait()
                gds[k][1].wait()

                @plsc.parallel_loop(0, 4)
                def _(g4):
                    i16 = g4 * 16 + lax.iota(jnp.int32, 16)
                    s16 = plsc.load_gather(r, [i16, jnp.full((16,), D + 1, jnp.int32)])
                    d16 = plsc.load_gather(dbuf, [i16, jnp.full((16,), dcol, jnp.int32)])
                    t = s16 + d16
                    t = jnp.where(t >= 0.0, t, 0.2 * t)
                    w_v[pl.ds(g4 * 16, 16)] = jnp.exp(t)

                @plsc.parallel_loop(0, CK)
                def _(j):
                    wj = plsc.load_gather(w_v, [jnp.zeros((16,), jnp.int32) + j])
                    for l in range(DA // 16):
                        r[j, pl.ds(l * 16, 16)] = r[j, pl.ds(l * 16, 16)] * wj

                sds[k] = pltpu.async_copy(
                    r, acc.at[edg_v.at[k, 1]], ssems[b], add=True
                )
                if k + 2 < GRP:
                    if k >= 1:
                        sds[k - 1].wait()
                    gds[k + 2] = gather(k + 2)
            sds[GRP - 2].wait()
            sds[GRP - 1].wait()
            return 0
        lax.fori_loop(0, ngrp, group, 0)
        plsc.subcore_barrier()
        pltpu.sync_copy(acc.at[pl.ds(sid * RPT, RPT)],
                        out_hbm.at[cid, pl.ds(sid * RPT, RPT)])
        plsc.subcore_barrier()

    phase(edl, NG_L, xwal, out_l, 0)
    phase(edg, NG_G, xwag, out_g, 1)


def _sc_accumulate(xwal, xwag, dt, edl, edg):
    mesh = plsc.VectorSubcoreMesh(core_axis_name="c", subcore_axis_name="s")
    return pl.kernel(
        _sc_body,
        out_type=[
            jax.ShapeDtypeStruct((2, NPAD, DA), jnp.float32),
            jax.ShapeDtypeStruct((2, NPAD, DA), jnp.float32),
        ],
        mesh=mesh,
        scratch_types=[
            pltpu.VMEM((CK, DA), jnp.float32),
            pltpu.VMEM((CK, DA), jnp.float32),
            pltpu.VMEM((CK, DA), jnp.float32),
            pltpu.VMEM((CK, 16), jnp.float32),
            pltpu.VMEM((CK, 16), jnp.float32),
            pltpu.VMEM((CK, 16), jnp.float32),
            pltpu.VMEM((GRP, 2, CK), jnp.int32),
            pltpu.VMEM((CK,), jnp.float32),
            pltpu.VMEM_SHARED((NPAD, DA), jnp.float32),
            pltpu.SemaphoreType.DMA,
            pltpu.SemaphoreType.DMA,
            pltpu.SemaphoreType.DMA,
            pltpu.SemaphoreType.DMA,
            pltpu.SemaphoreType.DMA,
            pltpu.SemaphoreType.DMA,
        ],
        compiler_params=pltpu.CompilerParams(
            needs_layout_passes=False, use_tc_tiling_on_sc=False
        ),
    )(xwal, xwag, dt, edl, edg)


def _combine_body(outl_ref, outg_ref, aux_ref, wc_ref, y_ref):
    al = outl_ref[0] + outl_ref[1]
    ag = outg_ref[0] + outg_ref[1]
    hl = al[:, :D] / (al[:, D:D + 1] + 1e-16) + aux_ref[0:1, :]
    hg = ag[:, :D] / (ag[:, D:D + 1] + 1e-16) + aux_ref[1:2, :]
    h = jnp.concatenate([hl, hg], axis=1)
    z = jnp.dot(h, wc_ref[...], preferred_element_type=jnp.float32)
    z = z + aux_ref[2:3, :]
    y_ref[...] = jnp.where(z > 0.0, z, jnp.exp(jnp.minimum(z, 0.0)) - 1.0)


def _combine(out_l, out_g, aux, W_comb):
    nblk = NPAD // BLK
    return pl.pallas_call(
        _combine_body,
        grid=(nblk,),
        in_specs=[
            pl.BlockSpec((2, BLK, DA), lambda i: (0, i, 0)),
            pl.BlockSpec((2, BLK, DA), lambda i: (0, i, 0)),
            pl.BlockSpec((8, D), lambda i: (0, 0)),
            pl.BlockSpec((2 * D, D), lambda i: (0, 0)),
        ],
        out_specs=pl.BlockSpec((BLK, D), lambda i: (i, 0)),
        out_shape=jax.ShapeDtypeStruct((NPAD, D), jnp.float32),
    )(out_l, out_g, aux, W_comb)


def kernel(x, edge_index_local, edge_index_global, W_local, att_src_local,
           att_dst_local, bias_local, W_global, att_src_global,
           att_dst_global, bias_global, W_comb, b_comb):
    x_pad = jnp.pad(x, ((0, NPAD - NN), (0, 0)))
    att8 = (
        jnp.zeros((8, D), jnp.float32)
        .at[0].set(att_src_local.reshape(D))
        .at[1].set(att_dst_local.reshape(D))
        .at[2].set(att_src_global.reshape(D))
        .at[3].set(att_dst_global.reshape(D))
    )
    xwal, xwag, dt = _precompute(x_pad, W_local, W_global, att8)

    def prep(e, epad, ngrp):
        e = jnp.pad(e, ((0, 0), (0, epad - e.shape[1])), constant_values=TRASH)
        src = e[0].reshape(NTILE, ngrp, GRP, CK)
        dst = e[1].reshape(NTILE, ngrp, GRP, CK)
        return jnp.stack([src, dst], axis=3)

    edl = prep(edge_index_local, EL_PAD, NG_L)
    edg = prep(edge_index_global, EG_PAD, NG_G)
    out_l, out_g = _sc_accumulate(xwal, xwag, dt, edl, edg)

    aux = (
        jnp.zeros((8, D), jnp.float32)
        .at[0].set(bias_local)
        .at[1].set(bias_global)
        .at[2].set(b_comb)
    )
    y = _combine(out_l, out_g, aux, W_comb)
    return y[:NN]
